# Initial kernel scaffold; baseline (speedup 1.0000x reference)
import functools

import jax
import jax.numpy as jnp
from jax import lax
from jax.experimental import pallas as pl
from jax.experimental.pallas import tpu as pltpu
from jax.experimental.pallas import tpu_sc as plsc

N = 10000
D = 128
E = 320000

NC = 2
NS = 16
NW = NC * NS
CH = 128
NCHUNK = 80
E_PAD = NW * NCHUNK * CH
N_PAD = 10016
RPT = N_PAD // NS

_mesh = plsc.VectorSubcoreMesh(core_axis_name="c", subcore_axis_name="s")


@functools.partial(
    pl.kernel,
    out_type=jax.ShapeDtypeStruct((NC, N_PAD, D), jnp.float32),
    mesh=_mesh,
    scratch_types=[
        pltpu.VMEM((NCHUNK, CH), jnp.int32),
        pltpu.VMEM((NCHUNK, CH), jnp.int32),
        pltpu.VMEM((CH, D), jnp.float32),
        pltpu.VMEM_SHARED((N_PAD, D), jnp.float32),
        pltpu.SemaphoreType.DMA,
    ],
)
def _sc_scatter_rows(h_hbm, src_hbm, dst_hbm, zeros_hbm, out_hbm,
                     src_v, dst_v, rows_v, agg_sh, sem):
    cid = lax.axis_index("c")
    sid = lax.axis_index("s")
    wid = cid * NS + sid
    pltpu.sync_copy(zeros_hbm, agg_sh.at[pl.ds(sid * RPT, RPT)])
    pltpu.sync_copy(src_hbm.at[wid], src_v)
    pltpu.sync_copy(dst_hbm.at[wid], dst_v)
    plsc.subcore_barrier()

    def body(j, carry):
        pltpu.async_copy(h_hbm.at[src_v.at[j]], rows_v, sem).wait()
        pltpu.sync_copy(rows_v, agg_sh.at[dst_v.at[j]], add=True)
        return carry

    lax.fori_loop(0, NCHUNK, body, 0)
    plsc.subcore_barrier()
    pltpu.sync_copy(agg_sh.at[pl.ds(sid * RPT, RPT)],
                    out_hbm.at[cid, pl.ds(sid * RPT, RPT)])


@functools.partial(
    pl.kernel,
    out_type=jax.ShapeDtypeStruct((NC, N_PAD, 16), jnp.float32),
    mesh=_mesh,
    scratch_types=[
        pltpu.VMEM((NCHUNK, CH), jnp.int32),
        pltpu.VMEM((CH, 16), jnp.float32),
        pltpu.VMEM_SHARED((N_PAD, 16), jnp.float32),
    ],
)
def _sc_degree(dst_hbm, ones_hbm, zeros_hbm, out_hbm, dst_v, ones_v, deg_sh):
    cid = lax.axis_index("c")
    sid = lax.axis_index("s")
    wid = cid * NS + sid
    pltpu.sync_copy(zeros_hbm, deg_sh.at[pl.ds(sid * RPT, RPT)])
    pltpu.sync_copy(dst_hbm.at[wid], dst_v)
    pltpu.sync_copy(ones_hbm, ones_v)
    plsc.subcore_barrier()

    def body(j, carry):
        pltpu.sync_copy(ones_v, deg_sh.at[dst_v.at[j]], add=True)
        return carry

    lax.fori_loop(0, NCHUNK, body, 0)
    plsc.subcore_barrier()
    pltpu.sync_copy(deg_sh.at[pl.ds(sid * RPT, RPT)],
                    out_hbm.at[cid, pl.ds(sid * RPT, RPT)])


def _dinv_from_partials(degp):
    deg = 1.0 + degp[0, :N, 0] + degp[1, :N, 0]
    return lax.rsqrt(deg)


def _tc_first_body(x_ref, degp_ref, w_ref, out_ref):
    dinv = _dinv_from_partials(degp_ref[...])
    h = jnp.dot(x_ref[...], w_ref[...], preferred_element_type=jnp.float32)
    out_ref[...] = dinv[:, None] * h


def _tc_mid_body(p_ref, hp_ref, degp_ref, b_ref, g_ref, be_ref, w_ref, out_ref):
    dinv = _dinv_from_partials(degp_ref[...])
    agg = p_ref[0, :N, :] + p_ref[1, :N, :] + hp_ref[...]
    z = dinv[:, None] * agg + b_ref[...]
    m = jnp.mean(z, axis=0)
    v = jnp.mean((z - m) ** 2, axis=0)
    y = g_ref[...] * (z - m) / jnp.sqrt(v + 1e-5) + be_ref[...]
    y = jnp.maximum(y, 0.0)
    h = jnp.dot(y, w_ref[...], preferred_element_type=jnp.float32)
    out_ref[...] = dinv[:, None] * h


def _tc_last_body(p_ref, hp_ref, degp_ref, b_ref, out_ref):
    dinv = _dinv_from_partials(degp_ref[...])
    agg = p_ref[0, :N, :] + p_ref[1, :N, :] + hp_ref[...]
    out_ref[...] = dinv[:, None] * agg + b_ref[...]


_out_nd = jax.ShapeDtypeStruct((N, D), jnp.float32)
_tc_first = pl.pallas_call(_tc_first_body, out_shape=_out_nd)
_tc_mid = pl.pallas_call(_tc_mid_body, out_shape=_out_nd)
_tc_last = pl.pallas_call(_tc_last_body, out_shape=_out_nd)


def kernel(x, edge_index, W1, b1, g1, be1, W2, b2, g2, be2, W3, b3, g3, be3,
           W4, b4):
    src = edge_index[0].astype(jnp.int32)
    dst = edge_index[1].astype(jnp.int32)
    pad = E_PAD - E
    src3 = jnp.concatenate([src, jnp.zeros((pad,), jnp.int32)])
    dst3 = jnp.concatenate([dst, jnp.full((pad,), N, jnp.int32)])
    src3 = src3.reshape(NW, NCHUNK, CH)
    dst3 = dst3.reshape(NW, NCHUNK, CH)

    zrows = jnp.zeros((RPT, D), jnp.float32)
    zdeg = jnp.zeros((RPT, 16), jnp.float32)
    ones = jnp.ones((CH, 16), jnp.float32)

    degp = _sc_degree(dst3, ones, zdeg)

    hp = _tc_first(x, degp, W1)
    p = _sc_scatter_rows(hp, src3, dst3, zrows)
    hp = _tc_mid(p, hp, degp, b1, g1, be1, W2)
    p = _sc_scatter_rows(hp, src3, dst3, zrows)
    hp = _tc_mid(p, hp, degp, b2, g2, be2, W3)
    p = _sc_scatter_rows(hp, src3, dst3, zrows)
    hp = _tc_mid(p, hp, degp, b3, g3, be3, W4)
    p = _sc_scatter_rows(hp, src3, dst3, zrows)
    return _tc_last(p, hp, degp, b4)


# speedup vs baseline: 6.3981x; 6.3981x over previous
import functools

import jax
import jax.numpy as jnp
from jax import lax
from jax.experimental import pallas as pl
from jax.experimental.pallas import tpu as pltpu
from jax.experimental.pallas import tpu_sc as plsc

N = 10000
D = 128
E = 320000

NC = 2
NS = 16
NW = NC * NS
CH = 128
NCHUNK = 80
E_PAD = NW * NCHUNK * CH
N_PAD = 10112
RPT = N_PAD // NS

_mesh = plsc.VectorSubcoreMesh(core_axis_name="c", subcore_axis_name="s")


@functools.partial(
    pl.kernel,
    out_type=jax.ShapeDtypeStruct((NC, N_PAD, D), jnp.float32),
    mesh=_mesh,
    scratch_types=[
        pltpu.VMEM((NCHUNK, CH), jnp.int32),
        pltpu.VMEM((NCHUNK, CH), jnp.int32),
        pltpu.VMEM((CH, D), jnp.float32),
        pltpu.VMEM_SHARED((N_PAD, D), jnp.float32),
        pltpu.SemaphoreType.DMA,
    ],
)
def _sc_scatter_rows(h_hbm, src_hbm, dst_hbm, zeros_hbm, out_hbm,
                     src_v, dst_v, rows_v, agg_sh, sem):
    cid = lax.axis_index("c")
    sid = lax.axis_index("s")
    wid = cid * NS + sid
    pltpu.sync_copy(zeros_hbm, agg_sh.at[pl.ds(sid * RPT, RPT)])
    pltpu.sync_copy(src_hbm.at[wid], src_v)
    pltpu.sync_copy(dst_hbm.at[wid], dst_v)
    plsc.subcore_barrier()

    def body(j, carry):
        pltpu.async_copy(h_hbm.at[src_v.at[j]], rows_v, sem).wait()
        pltpu.sync_copy(rows_v, agg_sh.at[dst_v.at[j]], add=True)
        return carry

    lax.fori_loop(0, NCHUNK, body, 0)
    plsc.subcore_barrier()
    pltpu.sync_copy(agg_sh.at[pl.ds(sid * RPT, RPT)],
                    out_hbm.at[cid, pl.ds(sid * RPT, RPT)])


@functools.partial(
    pl.kernel,
    out_type=jax.ShapeDtypeStruct((NC, N_PAD, 16), jnp.float32),
    mesh=_mesh,
    scratch_types=[
        pltpu.VMEM((NCHUNK, CH), jnp.int32),
        pltpu.VMEM((CH, 16), jnp.float32),
        pltpu.VMEM_SHARED((N_PAD, 16), jnp.float32),
    ],
)
def _sc_degree(dst_hbm, ones_hbm, zeros_hbm, out_hbm, dst_v, ones_v, deg_sh):
    cid = lax.axis_index("c")
    sid = lax.axis_index("s")
    wid = cid * NS + sid
    pltpu.sync_copy(zeros_hbm, deg_sh.at[pl.ds(sid * RPT, RPT)])
    pltpu.sync_copy(dst_hbm.at[wid], dst_v)
    pltpu.sync_copy(ones_hbm, ones_v)
    plsc.subcore_barrier()

    def body(j, carry):
        pltpu.sync_copy(ones_v, deg_sh.at[dst_v.at[j]], add=True)
        return carry

    lax.fori_loop(0, NCHUNK, body, 0)
    plsc.subcore_barrier()
    pltpu.sync_copy(deg_sh.at[pl.ds(sid * RPT, RPT)],
                    out_hbm.at[cid, pl.ds(sid * RPT, RPT)])


def _dinv_from_partials(degp):
    deg = 1.0 + degp[0, :N, 0] + degp[1, :N, 0]
    return lax.rsqrt(deg)


def _tc_first_body(x_ref, degp_ref, w_ref, out_ref):
    dinv = _dinv_from_partials(degp_ref[...])
    h = jnp.dot(x_ref[...], w_ref[...], preferred_element_type=jnp.float32)
    out_ref[...] = dinv[:, None] * h


def _tc_mid_body(p_ref, hp_ref, degp_ref, b_ref, g_ref, be_ref, w_ref, out_ref):
    dinv = _dinv_from_partials(degp_ref[...])
    agg = p_ref[0, :N, :] + p_ref[1, :N, :] + hp_ref[...]
    z = dinv[:, None] * agg + b_ref[...]
    m = jnp.mean(z, axis=0)
    v = jnp.mean((z - m) ** 2, axis=0)
    y = g_ref[...] * (z - m) / jnp.sqrt(v + 1e-5) + be_ref[...]
    y = jnp.maximum(y, 0.0)
    h = jnp.dot(y, w_ref[...], preferred_element_type=jnp.float32)
    out_ref[...] = dinv[:, None] * h


def _tc_last_body(p_ref, hp_ref, degp_ref, b_ref, out_ref):
    dinv = _dinv_from_partials(degp_ref[...])
    agg = p_ref[0, :N, :] + p_ref[1, :N, :] + hp_ref[...]
    out_ref[...] = dinv[:, None] * agg + b_ref[...]


_out_nd = jax.ShapeDtypeStruct((N, D), jnp.float32)
_tc_first = pl.pallas_call(_tc_first_body, out_shape=_out_nd)
_tc_mid = pl.pallas_call(_tc_mid_body, out_shape=_out_nd)
_tc_last = pl.pallas_call(_tc_last_body, out_shape=_out_nd)


def kernel(x, edge_index, W1, b1, g1, be1, W2, b2, g2, be2, W3, b3, g3, be3,
           W4, b4):
    src = edge_index[0].astype(jnp.int32)
    dst = edge_index[1].astype(jnp.int32)
    pad = E_PAD - E
    src3 = jnp.concatenate([src, jnp.zeros((pad,), jnp.int32)])
    dst3 = jnp.concatenate([dst, jnp.full((pad,), N, jnp.int32)])
    src3 = src3.reshape(NW, NCHUNK, CH)
    dst3 = dst3.reshape(NW, NCHUNK, CH)

    zrows = jnp.zeros((RPT, D), jnp.float32)
    zdeg = jnp.zeros((RPT, 16), jnp.float32)
    ones = jnp.ones((CH, 16), jnp.float32)

    degp = _sc_degree(dst3, ones, zdeg)

    hp = _tc_first(x, degp, W1)
    p = _sc_scatter_rows(hp, src3, dst3, zrows)
    hp = _tc_mid(p, hp, degp, b1, g1, be1, W2)
    p = _sc_scatter_rows(hp, src3, dst3, zrows)
    hp = _tc_mid(p, hp, degp, b2, g2, be2, W3)
    p = _sc_scatter_rows(hp, src3, dst3, zrows)
    hp = _tc_mid(p, hp, degp, b3, g3, be3, W4)
    p = _sc_scatter_rows(hp, src3, dst3, zrows)
    return _tc_last(p, hp, degp, b4)


# speedup vs baseline: 10.7035x; 1.6729x over previous
import functools

import jax
import jax.numpy as jnp
from jax import lax
from jax.experimental import pallas as pl
from jax.experimental.pallas import tpu as pltpu
from jax.experimental.pallas import tpu_sc as plsc

N = 10000
D = 128
E = 320000

NC = 2
NS = 16
NW = NC * NS
CH = 128
NCHUNK = 80
E_PAD = NW * NCHUNK * CH
N_PAD = 10112
RPT = N_PAD // NS

_mesh = plsc.VectorSubcoreMesh(core_axis_name="c", subcore_axis_name="s")


@functools.partial(
    pl.kernel,
    out_type=jax.ShapeDtypeStruct((NC, N_PAD, D), jnp.float32),
    mesh=_mesh,
    scratch_types=(
        [pltpu.VMEM((CH,), jnp.int32)] * 2
        + [pltpu.VMEM((CH, D), jnp.float32)]
        + [pltpu.VMEM_SHARED((N_PAD, D), jnp.float32)]
    ),
)
def _sc_scatter_rows(h_hbm, src_hbm, dst_hbm, zeros_hbm, out_hbm,
                     src_v, dst_v, row_v, agg_sh):
    cid = lax.axis_index("c")
    sid = lax.axis_index("s")
    wid = cid * NS + sid
    pltpu.sync_copy(zeros_hbm, agg_sh.at[pl.ds(sid * RPT, RPT)])
    plsc.subcore_barrier()

    def body(j, carry):
        pltpu.sync_copy(src_hbm.at[wid, j], src_v)
        pltpu.sync_copy(dst_hbm.at[wid, j], dst_v)
        pltpu.sync_copy(h_hbm.at[src_v], row_v)
        pltpu.sync_copy(row_v, agg_sh.at[dst_v], add=True)
        return carry

    lax.fori_loop(0, NCHUNK, body, 0)
    plsc.subcore_barrier()
    pltpu.sync_copy(agg_sh.at[pl.ds(sid * RPT, RPT)],
                    out_hbm.at[cid, pl.ds(sid * RPT, RPT)])


def _dinv_from_partials(degp):
    deg = 1.0 + degp[0, :N, 0] + degp[1, :N, 0]
    return lax.rsqrt(deg)


def _tc_first_body(x_ref, degp_ref, w_ref, out_ref):
    dinv = _dinv_from_partials(degp_ref[...])
    h = jnp.dot(x_ref[...], w_ref[...], preferred_element_type=jnp.float32)
    out_ref[...] = dinv[:, None] * h


def _tc_mid_body(p_ref, hp_ref, degp_ref, b_ref, g_ref, be_ref, w_ref, out_ref):
    dinv = _dinv_from_partials(degp_ref[...])
    agg = p_ref[0, :N, :] + p_ref[1, :N, :] + hp_ref[...]
    z = dinv[:, None] * agg + b_ref[...]
    m = jnp.mean(z, axis=0)
    v = jnp.mean((z - m) ** 2, axis=0)
    y = g_ref[...] * (z - m) / jnp.sqrt(v + 1e-5) + be_ref[...]
    y = jnp.maximum(y, 0.0)
    h = jnp.dot(y, w_ref[...], preferred_element_type=jnp.float32)
    out_ref[...] = dinv[:, None] * h


def _tc_last_body(p_ref, hp_ref, degp_ref, b_ref, out_ref):
    dinv = _dinv_from_partials(degp_ref[...])
    agg = p_ref[0, :N, :] + p_ref[1, :N, :] + hp_ref[...]
    out_ref[...] = dinv[:, None] * agg + b_ref[...]


_out_nd = jax.ShapeDtypeStruct((N, D), jnp.float32)
_tc_first = pl.pallas_call(_tc_first_body, out_shape=_out_nd)
_tc_mid = pl.pallas_call(_tc_mid_body, out_shape=_out_nd)
_tc_last = pl.pallas_call(_tc_last_body, out_shape=_out_nd)


def kernel(x, edge_index, W1, b1, g1, be1, W2, b2, g2, be2, W3, b3, g3, be3,
           W4, b4):
    src = edge_index[0].astype(jnp.int32)
    dst = edge_index[1].astype(jnp.int32)
    pad = E_PAD - E
    pid = jnp.arange(pad, dtype=jnp.int32)
    src3 = jnp.concatenate([src, pid % N])
    dst3 = jnp.concatenate([dst, N + pid % (N_PAD - N)])
    src3 = src3.reshape(NW, NCHUNK, CH)
    dst3 = dst3.reshape(NW, NCHUNK, CH)

    zrows = jnp.zeros((RPT, D), jnp.float32)

    degp = _sc_scatter_rows(jnp.ones((N, D), jnp.float32), src3, dst3, zrows)

    hp = _tc_first(x, degp, W1)
    p = _sc_scatter_rows(hp, src3, dst3, zrows)
    hp = _tc_mid(p, hp, degp, b1, g1, be1, W2)
    p = _sc_scatter_rows(hp, src3, dst3, zrows)
    hp = _tc_mid(p, hp, degp, b2, g2, be2, W3)
    p = _sc_scatter_rows(hp, src3, dst3, zrows)
    hp = _tc_mid(p, hp, degp, b3, g3, be3, W4)
    p = _sc_scatter_rows(hp, src3, dst3, zrows)
    return _tc_last(p, hp, degp, b4)


# speedup vs baseline: 11.9822x; 1.1195x over previous
import functools

import jax
import jax.numpy as jnp
from jax import lax
from jax.experimental import pallas as pl
from jax.experimental.pallas import tpu as pltpu
from jax.experimental.pallas import tpu_sc as plsc

N = 10000
D = 128
E = 320000

NC = 2
NS = 16
NW = NC * NS
CH = 128
NCHUNK = 80
E_PAD = NW * NCHUNK * CH
N_PAD = 10112
RPT = N_PAD // NS

_mesh = plsc.VectorSubcoreMesh(core_axis_name="c", subcore_axis_name="s")


@functools.partial(
    pl.kernel,
    out_type=jax.ShapeDtypeStruct((NC, N_PAD, D), jnp.float32),
    mesh=_mesh,
    scratch_types=(
        [pltpu.VMEM((CH,), jnp.int32)] * 2
        + [pltpu.VMEM((CH, D), jnp.float32)]
        + [pltpu.VMEM_SHARED((N_PAD, D), jnp.float32)]
    ),
)
def _sc_scatter_rows(h_hbm, src_hbm, dst_hbm, zeros_hbm, out_hbm,
                     src_v, dst_v, row_v, agg_sh):
    cid = lax.axis_index("c")
    sid = lax.axis_index("s")
    wid = cid * NS + sid
    pltpu.sync_copy(zeros_hbm, agg_sh.at[pl.ds(sid * RPT, RPT)])
    plsc.subcore_barrier()

    def body(j, carry):
        pltpu.sync_copy(src_hbm.at[wid, j], src_v)
        pltpu.sync_copy(dst_hbm.at[wid, j], dst_v)
        pltpu.sync_copy(h_hbm.at[src_v], row_v)
        pltpu.sync_copy(row_v, agg_sh.at[dst_v], add=True)
        return carry

    lax.fori_loop(0, NCHUNK, body, 0)
    plsc.subcore_barrier()
    pltpu.sync_copy(agg_sh.at[pl.ds(sid * RPT, RPT)],
                    out_hbm.at[cid, pl.ds(sid * RPT, RPT)])


@functools.partial(
    pl.kernel,
    out_type=jax.ShapeDtypeStruct((NC, N_PAD, D), jnp.float32),
    mesh=_mesh,
    scratch_types=(
        [pltpu.VMEM((CH,), jnp.int32)]
        + [pltpu.VMEM((CH, D), jnp.float32)]
        + [pltpu.VMEM_SHARED((N_PAD, D), jnp.float32)]
    ),
)
def _sc_degree_rows(dst_hbm, ones_hbm, zeros_hbm, out_hbm, dst_v, ones_v,
                    deg_sh):
    cid = lax.axis_index("c")
    sid = lax.axis_index("s")
    wid = cid * NS + sid
    pltpu.sync_copy(zeros_hbm, deg_sh.at[pl.ds(sid * RPT, RPT)])
    pltpu.sync_copy(ones_hbm, ones_v)
    plsc.subcore_barrier()

    def body(j, carry):
        pltpu.sync_copy(dst_hbm.at[wid, j], dst_v)
        pltpu.sync_copy(ones_v, deg_sh.at[dst_v], add=True)
        return carry

    lax.fori_loop(0, NCHUNK, body, 0)
    plsc.subcore_barrier()
    pltpu.sync_copy(deg_sh.at[pl.ds(sid * RPT, RPT)],
                    out_hbm.at[cid, pl.ds(sid * RPT, RPT)])


def _dinv_from_partials(degp):
    deg = 1.0 + degp[0, :N, 0] + degp[1, :N, 0]
    return lax.rsqrt(deg)


def _tc_first_body(x_ref, degp_ref, w_ref, out_ref):
    dinv = _dinv_from_partials(degp_ref[...])
    h = jnp.dot(x_ref[...], w_ref[...], preferred_element_type=jnp.float32)
    out_ref[...] = dinv[:, None] * h


def _tc_mid_body(p_ref, hp_ref, degp_ref, b_ref, g_ref, be_ref, w_ref, out_ref):
    dinv = _dinv_from_partials(degp_ref[...])
    agg = p_ref[0, :N, :] + p_ref[1, :N, :] + hp_ref[...]
    z = dinv[:, None] * agg + b_ref[...]
    m = jnp.mean(z, axis=0)
    v = jnp.mean((z - m) ** 2, axis=0)
    y = g_ref[...] * (z - m) / jnp.sqrt(v + 1e-5) + be_ref[...]
    y = jnp.maximum(y, 0.0)
    h = jnp.dot(y, w_ref[...], preferred_element_type=jnp.float32)
    out_ref[...] = dinv[:, None] * h


def _tc_last_body(p_ref, hp_ref, degp_ref, b_ref, out_ref):
    dinv = _dinv_from_partials(degp_ref[...])
    agg = p_ref[0, :N, :] + p_ref[1, :N, :] + hp_ref[...]
    out_ref[...] = dinv[:, None] * agg + b_ref[...]


_out_nd = jax.ShapeDtypeStruct((N, D), jnp.float32)
_tc_first = pl.pallas_call(_tc_first_body, out_shape=_out_nd)
_tc_mid = pl.pallas_call(_tc_mid_body, out_shape=_out_nd)
_tc_last = pl.pallas_call(_tc_last_body, out_shape=_out_nd)


def kernel(x, edge_index, W1, b1, g1, be1, W2, b2, g2, be2, W3, b3, g3, be3,
           W4, b4):
    src = edge_index[0].astype(jnp.int32)
    dst = edge_index[1].astype(jnp.int32)
    pad = E_PAD - E
    pid = jnp.arange(pad, dtype=jnp.int32)
    src3 = jnp.concatenate([src, pid % N])
    dst3 = jnp.concatenate([dst, N + pid % (N_PAD - N)])
    src3 = src3.reshape(NW, NCHUNK, CH)
    dst3 = dst3.reshape(NW, NCHUNK, CH)

    zrows = jnp.zeros((RPT, D), jnp.float32)

    degp = _sc_degree_rows(dst3, jnp.ones((CH, D), jnp.float32), zrows)

    hp = _tc_first(x, degp, W1)
    p = _sc_scatter_rows(hp, src3, dst3, zrows)
    hp = _tc_mid(p, hp, degp, b1, g1, be1, W2)
    p = _sc_scatter_rows(hp, src3, dst3, zrows)
    hp = _tc_mid(p, hp, degp, b2, g2, be2, W3)
    p = _sc_scatter_rows(hp, src3, dst3, zrows)
    hp = _tc_mid(p, hp, degp, b3, g3, be3, W4)
    p = _sc_scatter_rows(hp, src3, dst3, zrows)
    return _tc_last(p, hp, degp, b4)


# speedup vs baseline: 15.3298x; 1.2794x over previous
import functools

import jax
import jax.numpy as jnp
from jax import lax
from jax.experimental import pallas as pl
from jax.experimental.pallas import tpu as pltpu
from jax.experimental.pallas import tpu_sc as plsc

N = 10000
D = 128
E = 320000

NC = 2
NS = 16
NW = NC * NS
CH = 256
NCHUNK = 40
E_PAD = NW * NCHUNK * CH
N_PAD = 10112
RPT = N_PAD // NS

_mesh = plsc.VectorSubcoreMesh(core_axis_name="c", subcore_axis_name="s")


@functools.partial(
    pl.kernel,
    out_type=jax.ShapeDtypeStruct((NC, N_PAD, D), jnp.float32),
    mesh=_mesh,
    scratch_types=(
        [pltpu.VMEM((CH,), jnp.int32)] * 2
        + [pltpu.VMEM((CH, D), jnp.float32)]
        + [pltpu.VMEM_SHARED((N_PAD, D), jnp.float32)]
    ),
)
def _sc_scatter_rows(h_hbm, src_hbm, dst_hbm, zeros_hbm, out_hbm,
                     src_v, dst_v, row_v, agg_sh):
    cid = lax.axis_index("c")
    sid = lax.axis_index("s")
    wid = cid * NS + sid
    pltpu.sync_copy(zeros_hbm, agg_sh.at[pl.ds(sid * RPT, RPT)])
    plsc.subcore_barrier()

    def body(j, carry):
        pltpu.sync_copy(src_hbm.at[wid, j], src_v)
        pltpu.sync_copy(dst_hbm.at[wid, j], dst_v)
        pltpu.sync_copy(h_hbm.at[src_v], row_v)
        pltpu.sync_copy(row_v, agg_sh.at[dst_v], add=True)
        return carry

    lax.fori_loop(0, NCHUNK, body, 0)
    plsc.subcore_barrier()
    pltpu.sync_copy(agg_sh.at[pl.ds(sid * RPT, RPT)],
                    out_hbm.at[cid, pl.ds(sid * RPT, RPT)])


@functools.partial(
    pl.kernel,
    out_type=jax.ShapeDtypeStruct((NC, N_PAD, D), jnp.float32),
    mesh=_mesh,
    scratch_types=(
        [pltpu.VMEM((CH,), jnp.int32)]
        + [pltpu.VMEM((CH, D), jnp.float32)]
        + [pltpu.VMEM_SHARED((N_PAD, D), jnp.float32)]
    ),
)
def _sc_degree_rows(dst_hbm, ones_hbm, zeros_hbm, out_hbm, dst_v, ones_v,
                    deg_sh):
    cid = lax.axis_index("c")
    sid = lax.axis_index("s")
    wid = cid * NS + sid
    pltpu.sync_copy(zeros_hbm, deg_sh.at[pl.ds(sid * RPT, RPT)])
    pltpu.sync_copy(ones_hbm, ones_v)
    plsc.subcore_barrier()

    def body(j, carry):
        pltpu.sync_copy(dst_hbm.at[wid, j], dst_v)
        pltpu.sync_copy(ones_v, deg_sh.at[dst_v], add=True)
        return carry

    lax.fori_loop(0, NCHUNK, body, 0)
    plsc.subcore_barrier()
    pltpu.sync_copy(deg_sh.at[pl.ds(sid * RPT, RPT)],
                    out_hbm.at[cid, pl.ds(sid * RPT, RPT)])


def _dinv_from_partials(degp):
    deg = 1.0 + degp[0, :N, 0] + degp[1, :N, 0]
    return lax.rsqrt(deg)


def _tc_first_body(x_ref, degp_ref, w_ref, out_ref):
    dinv = _dinv_from_partials(degp_ref[...])
    h = jnp.dot(x_ref[...], w_ref[...], preferred_element_type=jnp.float32)
    out_ref[...] = dinv[:, None] * h


def _tc_mid_body(p_ref, hp_ref, degp_ref, b_ref, g_ref, be_ref, w_ref, out_ref):
    dinv = _dinv_from_partials(degp_ref[...])
    agg = p_ref[0, :N, :] + p_ref[1, :N, :] + hp_ref[...]
    z = dinv[:, None] * agg + b_ref[...]
    m = jnp.mean(z, axis=0)
    v = jnp.mean((z - m) ** 2, axis=0)
    y = g_ref[...] * (z - m) / jnp.sqrt(v + 1e-5) + be_ref[...]
    y = jnp.maximum(y, 0.0)
    h = jnp.dot(y, w_ref[...], preferred_element_type=jnp.float32)
    out_ref[...] = dinv[:, None] * h


def _tc_last_body(p_ref, hp_ref, degp_ref, b_ref, out_ref):
    dinv = _dinv_from_partials(degp_ref[...])
    agg = p_ref[0, :N, :] + p_ref[1, :N, :] + hp_ref[...]
    out_ref[...] = dinv[:, None] * agg + b_ref[...]


_out_nd = jax.ShapeDtypeStruct((N, D), jnp.float32)
_tc_first = pl.pallas_call(_tc_first_body, out_shape=_out_nd)
_tc_mid = pl.pallas_call(_tc_mid_body, out_shape=_out_nd)
_tc_last = pl.pallas_call(_tc_last_body, out_shape=_out_nd)


def kernel(x, edge_index, W1, b1, g1, be1, W2, b2, g2, be2, W3, b3, g3, be3,
           W4, b4):
    src = edge_index[0].astype(jnp.int32)
    dst = edge_index[1].astype(jnp.int32)
    pad = E_PAD - E
    pid = jnp.arange(pad, dtype=jnp.int32)
    src3 = jnp.concatenate([src, pid % N])
    dst3 = jnp.concatenate([dst, N + pid % (N_PAD - N)])
    src3 = src3.reshape(NW, NCHUNK, CH)
    dst3 = dst3.reshape(NW, NCHUNK, CH)

    zrows = jnp.zeros((RPT, D), jnp.float32)

    degp = _sc_degree_rows(dst3, jnp.ones((CH, D), jnp.float32), zrows)

    hp = _tc_first(x, degp, W1)
    p = _sc_scatter_rows(hp, src3, dst3, zrows)
    hp = _tc_mid(p, hp, degp, b1, g1, be1, W2)
    p = _sc_scatter_rows(hp, src3, dst3, zrows)
    hp = _tc_mid(p, hp, degp, b2, g2, be2, W3)
    p = _sc_scatter_rows(hp, src3, dst3, zrows)
    hp = _tc_mid(p, hp, degp, b3, g3, be3, W4)
    p = _sc_scatter_rows(hp, src3, dst3, zrows)
    return _tc_last(p, hp, degp, b4)


# speedup vs baseline: 20.0748x; 1.3095x over previous
import functools

import jax
import jax.numpy as jnp
from jax import lax
from jax.experimental import pallas as pl
from jax.experimental.pallas import tpu as pltpu
from jax.experimental.pallas import tpu_sc as plsc

N = 10000
D = 128
E = 320000

NC = 2
NS = 16
NW = NC * NS
CH = 128
NCHUNK = 80
CHD = 256
NCHD = 40
E_PAD = NW * NCHUNK * CH
N_PAD = 10112
RPT = N_PAD // NS

_mesh = plsc.VectorSubcoreMesh(core_axis_name="c", subcore_axis_name="s")


@functools.partial(
    pl.kernel,
    out_type=jax.ShapeDtypeStruct((NC, N_PAD, D), jnp.float32),
    mesh=_mesh,
    scratch_types=(
        [pltpu.VMEM((NCHUNK, CH), jnp.int32)]
        + [pltpu.VMEM((CH,), jnp.int32)] * 2
        + [pltpu.VMEM((CH, D), jnp.float32)] * 2
        + [pltpu.VMEM_SHARED((N_PAD, D), jnp.float32)]
        + [pltpu.SemaphoreType.DMA] * 6
    ),
)
def _sc_scatter_rows(h_hbm, src_hbm, dst_hbm, zeros_hbm, out_hbm,
                     srcs_v, d0, d1, r0, r1, agg_sh,
                     si0, si1, sg0, sg1, ss0, ss1):
    dv = (d0, d1)
    rv = (r0, r1)
    si = (si0, si1)
    sg = (sg0, sg1)
    ss = (ss0, ss1)
    cid = lax.axis_index("c")
    sid = lax.axis_index("s")
    wid = cid * NS + sid
    pltpu.sync_copy(zeros_hbm, agg_sh.at[pl.ds(sid * RPT, RPT)])
    pltpu.sync_copy(src_hbm.at[wid], srcs_v)
    plsc.subcore_barrier()

    def _idx(c, b):
        pltpu.async_copy(dst_hbm.at[wid, c], dv[b], si[b])

    def _idx_wait(b):
        pltpu.make_async_copy(dst_hbm.at[0, 0], dv[b], si[b]).wait()

    def _gather(c, b):
        pltpu.async_copy(h_hbm.at[srcs_v.at[c]], rv[b], sg[b])

    def _gather_wait(b):
        pltpu.make_async_copy(h_hbm.at[srcs_v.at[0]], rv[b], sg[b]).wait()

    def _scatter(b):
        pltpu.async_copy(rv[b], agg_sh.at[dv[b]], ss[b], add=True)

    def _scatter_wait(b):
        pltpu.make_async_copy(rv[b], agg_sh.at[dv[b]], ss[b]).wait()

    for b in range(2):
        _idx(b, b)
        _gather(b, b)
    for b in range(2):
        _gather_wait(b)
        _idx_wait(b)
        _scatter(b)

    def body(i, carry):
        for b in range(2):
            c = 2 * i + b
            _scatter_wait(b)
            _idx(c, b)
            _gather(c, b)
            _gather_wait(b)
            _idx_wait(b)
            _scatter(b)
        return carry

    lax.fori_loop(1, NCHUNK // 2, body, 0)
    for b in range(2):
        _scatter_wait(b)
    plsc.subcore_barrier()
    pltpu.sync_copy(agg_sh.at[pl.ds(sid * RPT, RPT)],
                    out_hbm.at[cid, pl.ds(sid * RPT, RPT)])


@functools.partial(
    pl.kernel,
    out_type=jax.ShapeDtypeStruct((NC, N_PAD, D), jnp.float32),
    mesh=_mesh,
    scratch_types=(
        [pltpu.VMEM((CHD,), jnp.int32)]
        + [pltpu.VMEM((CHD, D), jnp.float32)]
        + [pltpu.VMEM_SHARED((N_PAD, D), jnp.float32)]
    ),
)
def _sc_degree_rows(dst_hbm, ones_hbm, zeros_hbm, out_hbm, dst_v, ones_v,
                    deg_sh):
    cid = lax.axis_index("c")
    sid = lax.axis_index("s")
    wid = cid * NS + sid
    pltpu.sync_copy(zeros_hbm, deg_sh.at[pl.ds(sid * RPT, RPT)])
    pltpu.sync_copy(ones_hbm, ones_v)
    plsc.subcore_barrier()

    def body(j, carry):
        pltpu.sync_copy(dst_hbm.at[wid, j], dst_v)
        pltpu.sync_copy(ones_v, deg_sh.at[dst_v], add=True)
        return carry

    lax.fori_loop(0, NCHD, body, 0)
    plsc.subcore_barrier()
    pltpu.sync_copy(deg_sh.at[pl.ds(sid * RPT, RPT)],
                    out_hbm.at[cid, pl.ds(sid * RPT, RPT)])


def _dinv_from_partials(degp):
    deg = 1.0 + degp[0, :N, 0] + degp[1, :N, 0]
    return lax.rsqrt(deg)


def _tc_first_body(x_ref, degp_ref, w_ref, out_ref):
    dinv = _dinv_from_partials(degp_ref[...])
    h = jnp.dot(x_ref[...], w_ref[...], preferred_element_type=jnp.float32)
    out_ref[...] = dinv[:, None] * h


def _tc_mid_body(p_ref, hp_ref, degp_ref, b_ref, g_ref, be_ref, w_ref, out_ref):
    dinv = _dinv_from_partials(degp_ref[...])
    agg = p_ref[0, :N, :] + p_ref[1, :N, :] + hp_ref[...]
    z = dinv[:, None] * agg + b_ref[...]
    m = jnp.mean(z, axis=0)
    v = jnp.mean((z - m) ** 2, axis=0)
    y = g_ref[...] * (z - m) / jnp.sqrt(v + 1e-5) + be_ref[...]
    y = jnp.maximum(y, 0.0)
    h = jnp.dot(y, w_ref[...], preferred_element_type=jnp.float32)
    out_ref[...] = dinv[:, None] * h


def _tc_last_body(p_ref, hp_ref, degp_ref, b_ref, out_ref):
    dinv = _dinv_from_partials(degp_ref[...])
    agg = p_ref[0, :N, :] + p_ref[1, :N, :] + hp_ref[...]
    out_ref[...] = dinv[:, None] * agg + b_ref[...]


_out_nd = jax.ShapeDtypeStruct((N, D), jnp.float32)
_tc_first = pl.pallas_call(_tc_first_body, out_shape=_out_nd)
_tc_mid = pl.pallas_call(_tc_mid_body, out_shape=_out_nd)
_tc_last = pl.pallas_call(_tc_last_body, out_shape=_out_nd)


def kernel(x, edge_index, W1, b1, g1, be1, W2, b2, g2, be2, W3, b3, g3, be3,
           W4, b4):
    src = edge_index[0].astype(jnp.int32)
    dst = edge_index[1].astype(jnp.int32)
    pad = E_PAD - E
    pid = jnp.arange(pad, dtype=jnp.int32)
    src3 = jnp.concatenate([src, pid % N])
    dst3 = jnp.concatenate([dst, N + pid % (N_PAD - N)])
    src3 = src3.reshape(NW, NCHUNK, CH)
    dst3 = dst3.reshape(NW, NCHUNK, CH)

    zrows = jnp.zeros((RPT, D), jnp.float32)

    degp = _sc_degree_rows(dst3.reshape(NW, NCHD, CHD),
                           jnp.ones((CHD, D), jnp.float32), zrows)

    hp = _tc_first(x, degp, W1)
    p = _sc_scatter_rows(hp, src3, dst3, zrows)
    hp = _tc_mid(p, hp, degp, b1, g1, be1, W2)
    p = _sc_scatter_rows(hp, src3, dst3, zrows)
    hp = _tc_mid(p, hp, degp, b2, g2, be2, W3)
    p = _sc_scatter_rows(hp, src3, dst3, zrows)
    hp = _tc_mid(p, hp, degp, b3, g3, be3, W4)
    p = _sc_scatter_rows(hp, src3, dst3, zrows)
    return _tc_last(p, hp, degp, b4)
